# baseline (speedup 1.0000x reference)
import functools

import jax
import jax.numpy as jnp
from jax import lax
from jax.experimental import pallas as pl
from jax.experimental.pallas import tpu as pltpu
from jax.experimental.pallas import tpu_sc as plsc

_THRESHOLDS = (
    60 * 1000,
    5 * 60 * 1000,
    30 * 60 * 1000,
    120 * 60 * 1000,
    24 * 60 * 60 * 1000,
    7 * 24 * 60 * 60 * 1000,
)
_DIM = 64
_LANES = 16


@functools.lru_cache(maxsize=None)
def _build_sc_call(B: int):
    info = plsc.get_sparse_core_info()
    nw = info.num_cores * info.num_subcores
    assert B % nw == 0
    b_per_w = B // nw
    chunk = 1024
    assert b_per_w % chunk == 0
    n_chunks = b_per_w // chunk

    mesh = plsc.VectorSubcoreMesh(core_axis_name="c", subcore_axis_name="s")

    @functools.partial(
        pl.kernel,
        mesh=mesh,
        compiler_params=pltpu.CompilerParams(use_tc_tiling_on_sc=False),
        out_type=jax.ShapeDtypeStruct((B, _DIM), jnp.float32),
        scratch_types=[
            pltpu.VMEM((b_per_w,), jnp.int32),
            pltpu.VMEM((chunk,), jnp.int32),
            pltpu.VMEM((chunk, _DIM), jnp.float32),
            pltpu.SemaphoreType.DMA,
        ],
    )
    def sc_kernel(delta_hbm, w_hbm, out_hbm, delta_v, idx_v, rows_v, sem):
        wid = lax.axis_index("s") * info.num_cores + lax.axis_index("c")
        base = wid * b_per_w
        pltpu.sync_copy(delta_hbm.at[pl.ds(base, b_per_w)], delta_v)

        for c in range(n_chunks):
            def bucket_body(i, carry, c=c):
                d = delta_v[pl.ds(c * chunk + i * _LANES, _LANES)]
                b = jnp.zeros((_LANES,), jnp.int32)
                for t in _THRESHOLDS:
                    b = b + jnp.where(d >= t, 1, 0).astype(jnp.int32)
                idx_v[pl.ds(i * _LANES, _LANES)] = b
                return carry

            lax.fori_loop(0, chunk // _LANES, bucket_body, 0)
            pltpu.async_copy(w_hbm.at[idx_v], rows_v, sem).wait()
            pltpu.sync_copy(rows_v, out_hbm.at[pl.ds(base + c * chunk, chunk)])

    return sc_kernel


def kernel(time_delta_ms, W):
    n, t = time_delta_ms.shape
    flat = time_delta_ms.reshape(n * t)
    out = _build_sc_call(n * t)(flat, W)
    return out.reshape(n, t, _DIM)
